# baseline (speedup 1.0000x reference)
import functools

import jax
import jax.numpy as jnp
from jax import lax
from jax.experimental import pallas as pl
from jax.experimental.pallas import tpu as pltpu
from jax.experimental.pallas import tpu_sc as plsc

_GAMMA = 12.0
_GAMMA_TYPE = 6.0
_HID = 64
_TDIM = 32
_C = 32
_NIDX = 12
_WB = 400
_NB = 200


@functools.cache
def _build_relayout(E, R):
  E2, E4, R2, R4 = E // 2, E // 4, R // 2, R // 4
  grid = E2 // _WB
  assert grid == R2 // _WB == E4 // _NB == R4 // _NB
  f32 = jnp.float32

  def body(e0, e1, r0, r1, n0, n1,
           t0, t1, t2, t3, y0, y1, y2, y3, z0, z1, z2, z3,
           ent_o, rel_o, nv_o, typ_o, rtyp_o, nvt_o):
    for (a, b), o in (((e0, e1), ent_o), ((r0, r1), rel_o), ((n0, n1), nv_o)):
      o[:, 0:_HID] = a[...]
      o[:, _HID:2 * _HID] = b[...]
    for (a, b, c, d), o in (((t0, t1, t2, t3), typ_o),
                            ((y0, y1, y2, y3), rtyp_o),
                            ((z0, z1, z2, z3), nvt_o)):
      o[:, 0:_TDIM] = a[...]
      o[:, _TDIM:2 * _TDIM] = b[...]
      o[:, 2 * _TDIM:3 * _TDIM] = c[...]
      o[:, 3 * _TDIM:4 * _TDIM] = d[...]

  def wspec(k):
    return pl.BlockSpec((_WB, _HID), lambda i, k=k: (i + k * grid, 0))

  def nspec(k):
    return pl.BlockSpec((_NB, _TDIM), lambda i, k=k: (i + k * grid, 0))

  return pl.pallas_call(
      body,
      grid=(grid,),
      in_specs=([wspec(0), wspec(1)] * 3 + [nspec(k) for k in range(4)] * 3),
      out_specs=[pl.BlockSpec((_WB, 2 * _HID), lambda i: (i, 0))] * 3
      + [pl.BlockSpec((_NB, 4 * _TDIM), lambda i: (i, 0))] * 3,
      out_shape=[jax.ShapeDtypeStruct((E2, 2 * _HID), f32),
                 jax.ShapeDtypeStruct((R2, 2 * _HID), f32),
                 jax.ShapeDtypeStruct((R2, 2 * _HID), f32),
                 jax.ShapeDtypeStruct((E4, 4 * _TDIM), f32),
                 jax.ShapeDtypeStruct((R4, 4 * _TDIM), f32),
                 jax.ShapeDtypeStruct((R4, 4 * _TDIM), f32)],
      compiler_params=pltpu.CompilerParams(
          dimension_semantics=("arbitrary",)),
  )


@functools.cache
def _build_sc(B):
  info = plsc.get_sparse_core_info()
  NC, NS, L = info.num_cores, info.num_subcores, info.num_lanes
  NW = NC * NS
  assert B % (NW * _C) == 0
  per_w = B // NW
  n_chunks = per_w // _C
  assert n_chunks % 2 == 0
  groups = _C // L
  f32 = jnp.float32
  i32 = jnp.int32
  mesh = plsc.VectorSubcoreMesh(core_axis_name="c", subcore_axis_name="s")

  def _set():
    return [
        pltpu.VMEM((_NIDX * _C,), i32),
        pltpu.VMEM((2 * _C, 128), f32),
        pltpu.VMEM((2 * _C, 128), f32),
        pltpu.VMEM((_C, 128), f32),
        pltpu.VMEM((_C, 128), f32),
        pltpu.VMEM((_C, 128), f32),
        pltpu.VMEM((_C, 128), f32),
        pltpu.SemaphoreType.DMA,
    ]

  @functools.partial(
      pl.kernel,
      mesh=mesh,
      compiler_params=pltpu.CompilerParams(
          needs_layout_passes=False,
          disable_bounds_checks=True,
      ),
      out_type=[jax.ShapeDtypeStruct((B,), f32),
                jax.ShapeDtypeStruct((B,), f32)],
      scratch_types=(
          _set() + _set()
          + [
              pltpu.VMEM((per_w,), f32),
              pltpu.VMEM((per_w,), f32),
          ]),
  )
  def kge(pack_hbm, ent_hbm, rel_hbm, typ_hbm, rtyp_hbm, nv_hbm, nvt_hbm,
          score_hbm, scoret_hbm, *scratch):
    set0 = scratch[0:8]
    set1 = scratch[8:16]
    sc_v, sct_v = scratch[16:18]
    wid = lax.axis_index("s") * NC + lax.axis_index("c")
    base = wid * per_w

    def copies(bufs):
      idx_v, ent_v, typ_v, rel_v, nv_v, rtyp_v, nvt_v, sem = bufs
      ht2 = idx_v.at[pl.ds(0, 2 * _C)]
      ht4 = idx_v.at[pl.ds(2 * _C, 2 * _C)]
      r2 = idx_v.at[pl.ds(4 * _C, _C)]
      r4 = idx_v.at[pl.ds(5 * _C, _C)]
      return [
          pltpu.make_async_copy(ent_hbm.at[ht2], ent_v, sem),
          pltpu.make_async_copy(typ_hbm.at[ht4], typ_v, sem),
          pltpu.make_async_copy(rel_hbm.at[r2], rel_v, sem),
          pltpu.make_async_copy(nv_hbm.at[r2], nv_v, sem),
          pltpu.make_async_copy(rtyp_hbm.at[r4], rtyp_v, sem),
          pltpu.make_async_copy(nvt_hbm.at[r4], nvt_v, sem),
      ]

    def start_chunk(bufs, ci):
      idx_v = bufs[0]
      gchunk = wid * n_chunks + ci
      pltpu.sync_copy(pack_hbm.at[pl.ds(gchunk * (_NIDX * _C), _NIDX * _C)],
                      idx_v)
      for cp in copies(bufs):
        cp.start()

    def wait_chunk(bufs):
      for cp in copies(bufs):
        cp.wait()

    def compute_chunk(bufs, ci):
      idx_v, ent_v, typ_v, rel_v, nv_v, rtyp_v, nvt_v, _ = bufs
      lane = lax.iota(i32, L)

      def rsum(x):
        return jnp.broadcast_to(jnp.sum(x), (L,))

      def group_body(g, carry):
        score_acc = jnp.zeros((L,), f32)
        scoret_acc = jnp.zeros((L,), f32)
        hoffv = idx_v[pl.ds(6 * _C + g * L, L)]
        toffv = idx_v[pl.ds(7 * _C + g * L, L)]
        roffv = idx_v[pl.ds(8 * _C + g * L, L)]
        h4ov = idx_v[pl.ds(9 * _C + g * L, L)]
        t4ov = idx_v[pl.ds(10 * _C + g * L, L)]
        r4ov = idx_v[pl.ds(11 * _C + g * L, L)]
        for k in range(L):
          i = g * L + k
          hoff = hoffv[k]
          toff = toffv[k]
          roff = roffv[k]
          h4o = h4ov[k]
          t4o = t4ov[k]
          r4o = r4ov[k]

          hs = [ent_v[i, pl.ds(hoff + 16 * q, 16)] for q in range(4)]
          ts = [ent_v[_C + i, pl.ds(toff + 16 * q, 16)] for q in range(4)]
          rs = [rel_v[i, pl.ds(roff + 16 * q, 16)] for q in range(4)]
          vs = [nv_v[i, pl.ds(roff + 16 * q, 16)] for q in range(4)]
          hv = rsum((hs[0] * vs[0] + hs[1] * vs[1])
                    + (hs[2] * vs[2] + hs[3] * vs[3]))
          tv = rsum((ts[0] * vs[0] + ts[1] * vs[1])
                    + (ts[2] * vs[2] + ts[3] * vs[3]))
          vv = rsum((vs[0] * vs[0] + vs[1] * vs[1])
                    + (vs[2] * vs[2] + vs[3] * vs[3]))
          c = (tv - hv) / jnp.maximum(vv, 1e-24)
          s4 = [jnp.abs(hs[q] + rs[q] - ts[q] + c * vs[q]) for q in range(4)]
          score = _GAMMA - rsum((s4[0] + s4[1]) + (s4[2] + s4[3]))

          h2s = [typ_v[i, pl.ds(h4o + 16 * q, 16)] for q in range(2)]
          t2s = [typ_v[_C + i, pl.ds(t4o + 16 * q, 16)] for q in range(2)]
          r2s = [rtyp_v[i, pl.ds(r4o + 16 * q, 16)] for q in range(2)]
          v2s = [nvt_v[i, pl.ds(r4o + 16 * q, 16)] for q in range(2)]
          hv2 = rsum(h2s[0] * v2s[0] + h2s[1] * v2s[1])
          tv2 = rsum(t2s[0] * v2s[0] + t2s[1] * v2s[1])
          vv2 = rsum(v2s[0] * v2s[0] + v2s[1] * v2s[1])
          c2 = (tv2 - hv2) / jnp.maximum(vv2, 1e-24)
          s2 = [jnp.abs(h2s[q] + r2s[q] - t2s[q] + c2 * v2s[q])
                for q in range(2)]
          score_t = _GAMMA_TYPE - rsum(s2[0] + s2[1])

          score_acc = jnp.where(lane == k, score, score_acc)
          scoret_acc = jnp.where(lane == k, score_t, scoret_acc)

        out_off = ci * _C + g * L
        sc_v[pl.ds(out_off, L)] = score_acc
        sct_v[pl.ds(out_off, L)] = scoret_acc
        return carry

      lax.fori_loop(0, groups, group_body, 0)

    start_chunk(set0, 0)

    def chunk_pair(ci2, carry):
      ci = ci2 * 2
      wait_chunk(set0)
      start_chunk(set1, ci + 1)
      compute_chunk(set0, ci)
      wait_chunk(set1)

      @pl.when(ci + 2 < n_chunks)
      def _():
        start_chunk(set0, ci + 2)

      compute_chunk(set1, ci + 1)
      return carry

    lax.fori_loop(0, n_chunks // 2, chunk_pair, 0)
    pltpu.sync_copy(sc_v, score_hbm.at[pl.ds(base, per_w)])
    pltpu.sync_copy(sct_v, scoret_hbm.at[pl.ds(base, per_w)])

  return kge


def kernel(sample, entity_embedding, relation_embedding, type_embedding,
           reltype_embedding, norm_vector_embedding, norm_vectortype_embedding):
  B = sample.shape[0]
  E = entity_embedding.shape[0]
  R = relation_embedding.shape[0]
  E2, E4, R2, R4 = E // 2, E // 4, R // 2, R // 4

  ent2, rel2, nv2, typ4, rtyp4, nvt4 = _build_relayout(E, R)(
      entity_embedding, entity_embedding,
      relation_embedding, relation_embedding,
      norm_vector_embedding, norm_vector_embedding,
      type_embedding, type_embedding, type_embedding, type_embedding,
      reltype_embedding, reltype_embedding, reltype_embedding,
      reltype_embedding,
      norm_vectortype_embedding, norm_vectortype_embedding,
      norm_vectortype_embedding, norm_vectortype_embedding)

  h = sample[:, 0]
  r = sample[:, 1]
  t = sample[:, 2]
  def half(x, n):
    q = x >= n
    return jnp.where(q, x - n, x), jnp.where(q, _HID, 0).astype(jnp.int32)

  def quarter(x, n):
    q = x // n
    return x - q * n, (q * _TDIM).astype(jnp.int32)

  h2, hoff = half(h, E2)
  t2, toff = half(t, E2)
  r2, roff = half(r, R2)
  h4, h4o = quarter(h, E4)
  t4, t4o = quarter(t, E4)
  r4, r4o = quarter(r, R4)
  idx12 = jnp.stack([h2, t2, h4, t4, r2, r4, hoff, toff, roff, h4o, t4o, r4o])
  pack = idx12.reshape(_NIDX, B // _C, _C).transpose(1, 0, 2).reshape(-1)

  score, score_type = _build_sc(B)(
      pack, ent2, rel2, typ4, rtyp4, nv2, nvt4)
  return score.reshape(B, 1), score_type.reshape(B, 1)
